# baseline (speedup 1.0000x reference)
import functools

import jax
import jax.numpy as jnp
from jax import lax
from jax.experimental import pallas as pl
from jax.experimental.pallas import tpu as pltpu
from jax.experimental.pallas import tpu_sc as plsc

N = 10000
E = 320000
F_IN = 128
D = 64
EDGE_DIM = 16
G = 64

NC = 2
NS = 16
NW = NC * NS
CH = 80
EPW_G = 2 * E // NW
NCH_G = EPW_G // CH
GBUF = 5
CHS = 40
EPW_S = E // NW
NCH_S = EPW_S // CHS
SBUF = 10
MW = 72
NPAD = 10240
RPS = NPAD // NS
ZR = 8


def _worker_id():
    return lax.axis_index("s") * NC + lax.axis_index("c")


def _make_gather(dt):
    mesh = plsc.VectorSubcoreMesh(core_axis_name="c", subcore_axis_name="s")

    @functools.partial(
        pl.kernel,
        out_type=jax.ShapeDtypeStruct((2 * E, dt), jnp.float32),
        mesh=mesh,
        scratch_types=[
            pltpu.VMEM((NCH_G, CH), jnp.int32),
            pltpu.VMEM((GBUF, CH, dt), jnp.float32),
            pltpu.SemaphoreType.DMA((GBUF,)),
        ],
    )
    def gather_k(table_hbm, idx_hbm, out_hbm, idx_v, rows_v, sems):
        w = _worker_id()
        pltpu.sync_copy(idx_hbm.at[w], idx_v)
        ebase = w * EPW_G
        for b in range(GBUF):
            pltpu.make_async_copy(
                table_hbm.at[idx_v.at[b]], rows_v.at[b], sems.at[b]
            ).start()

        def body(t, _):
            for b in range(GBUF):
                i = t * GBUF + b
                pltpu.make_async_copy(
                    table_hbm.at[idx_v.at[i]], rows_v.at[b], sems.at[b]
                ).wait()
                pltpu.sync_copy(rows_v.at[b], out_hbm.at[pl.ds(ebase + i * CH, CH)])
                nxt = i + GBUF

                @pl.when(nxt < NCH_G)
                def _start():
                    pltpu.make_async_copy(
                        table_hbm.at[idx_v.at[nxt]], rows_v.at[b], sems.at[b]
                    ).start()

            return 0

        lax.fori_loop(0, NCH_G // GBUF, body, 0)

    return gather_k


def _make_scatter_add():
    mesh = plsc.VectorSubcoreMesh(core_axis_name="c", subcore_axis_name="s")

    @functools.partial(
        pl.kernel,
        out_type=jax.ShapeDtypeStruct((NC, NPAD, MW), jnp.float32),
        mesh=mesh,
        scratch_types=[
            pltpu.VMEM_SHARED((NPAD, MW), jnp.float32),
            pltpu.VMEM((NCH_S, CHS), jnp.int32),
            pltpu.VMEM((SBUF, CHS, MW), jnp.float32),
            pltpu.VMEM((ZR, MW), jnp.float32),
            pltpu.SemaphoreType.DMA((SBUF,)),
        ],
    )
    def scatter_k(msg_hbm, dst_hbm, out_hbm, acc_sh, dst_v, msg_v, zbuf, sems):
        c = lax.axis_index("c")
        s = lax.axis_index("s")
        w = s * NC + c

        def zrow(r, _):
            for cc in range(MW // 16):
                zbuf[r, pl.ds(cc * 16, 16)] = jnp.zeros((16,), jnp.float32)
            return 0

        lax.fori_loop(0, ZR, zrow, 0)
        for j in range(RPS // ZR):
            pltpu.sync_copy(zbuf, acc_sh.at[pl.ds(s * RPS + j * ZR, ZR)])
        plsc.subcore_barrier()

        pltpu.sync_copy(dst_hbm.at[w], dst_v)
        for b in range(SBUF):
            pltpu.make_async_copy(
                msg_hbm.at[pl.ds(w * EPW_S + b * CHS, CHS)], msg_v.at[b], sems.at[b]
            ).start()

        def body(t, _):
            for b in range(SBUF):
                i = t * SBUF + b
                pltpu.make_async_copy(
                    msg_hbm.at[pl.ds(w * EPW_S + i * CHS, CHS)], msg_v.at[b], sems.at[b]
                ).wait()
                pltpu.sync_copy(msg_v.at[b], acc_sh.at[dst_v.at[i]], add=True)
                nxt = i + SBUF

                @pl.when(nxt < NCH_S)
                def _start():
                    pltpu.make_async_copy(
                        msg_hbm.at[pl.ds(w * EPW_S + nxt * CHS, CHS)],
                        msg_v.at[b],
                        sems.at[b],
                    ).start()

            return 0

        lax.fori_loop(0, NCH_S // SBUF, body, 0)
        plsc.subcore_barrier()
        pltpu.sync_copy(acc_sh.at[pl.ds(s * RPS, RPS)], out_hbm.at[c, pl.ds(s * RPS, RPS)])

    return scatter_k


_gather_all = _make_gather(2 * D)
_scatter_add = _make_scatter_add()


_BN = 1000
_BEDGE = 4000


def _tables_tc(h, wall, ball):
    f = h.shape[1]
    nb = N // _BN

    def body(h_ref, w_ref, b_ref, out_ref):
        out_ref[...] = h_ref[...] @ w_ref[0] + b_ref[0]

    return pl.pallas_call(
        body,
        grid=(2 * nb,),
        in_specs=[
            pl.BlockSpec((_BN, f), lambda i: (i % nb, 0)),
            pl.BlockSpec((1, f, 2 * D), lambda i: (i // nb, 0, 0)),
            pl.BlockSpec((1, 1, 2 * D), lambda i: (i // nb, 0, 0)),
        ],
        out_specs=pl.BlockSpec((_BN, 2 * D), lambda i: (i, 0)),
        out_shape=jax.ShapeDtypeStruct((2 * N, 2 * D), jnp.float32),
    )(h, wall, ball)


def _edge_tc(kvj, qd, ea, we, be):
    def body(kvj_ref, qd_ref, ea_ref, we_ref, be_ref, out_ref):
        e = ea_ref[...] @ we_ref[...] + be_ref[...]
        kj = kvj_ref[:, :D] + e
        alpha = jnp.sum(qd_ref[:, :D] * kj, axis=1, keepdims=True) * 0.125
        ex = jnp.exp(alpha)
        out_ref[:, :D] = (kvj_ref[:, D:] + e) * ex
        out_ref[:, D:D + 1] = ex
        out_ref[:, D + 1:] = jnp.zeros((out_ref.shape[0], MW - D - 1), jnp.float32)

    nbe = E // _BEDGE
    return pl.pallas_call(
        body,
        grid=(nbe,),
        in_specs=[
            pl.BlockSpec((_BEDGE, 2 * D), lambda i: (i, 0)),
            pl.BlockSpec((_BEDGE, 2 * D), lambda i: (i + nbe, 0)),
            pl.BlockSpec((_BEDGE, EDGE_DIM), lambda i: (i, 0)),
            pl.BlockSpec((EDGE_DIM, D), lambda i: (0, 0)),
            pl.BlockSpec((1, D), lambda i: (0, 0)),
        ],
        out_specs=pl.BlockSpec((_BEDGE, MW), lambda i: (i, 0)),
        out_shape=jax.ShapeDtypeStruct((E, MW), jnp.float32),
    )(kvj, qd, ea, we, be)


def _combine_tc(accs, h, ws, bs, m_prev, first, apply_elu):
    f = h.shape[1]

    def body(*refs):
        if first:
            acc_ref, h_ref, ws_ref, bs_ref, h_out, m_out = refs
        else:
            acc_ref, h_ref, ws_ref, bs_ref, m_ref, h_out, m_out = refs
        acc = acc_ref[0] + acc_ref[1]
        den = acc[:, D:D + 1]
        conv = acc[:, :D] / (den + 1e-16) + h_ref[...] @ ws_ref[...] + bs_ref[...]
        if apply_elu:
            conv = jnp.where(conv > 0, conv, jnp.exp(conv) - 1.0)
        h_out[...] = conv
        if first:
            m_out[...] = conv
        else:
            m_out[...] = jnp.maximum(m_ref[...], conv)

    in_specs = [
        pl.BlockSpec((NC, _BN, MW), lambda i: (0, i, 0)),
        pl.BlockSpec((_BN, f), lambda i: (i, 0)),
        pl.BlockSpec((f, D), lambda i: (0, 0)),
        pl.BlockSpec((1, D), lambda i: (0, 0)),
    ]
    args = [accs, h, ws, bs]
    if not first:
        in_specs.append(pl.BlockSpec((_BN, D), lambda i: (i, 0)))
        args.append(m_prev)

    return pl.pallas_call(
        body,
        grid=(N // _BN,),
        in_specs=in_specs,
        out_specs=[
            pl.BlockSpec((_BN, D), lambda i: (i, 0)),
            pl.BlockSpec((_BN, D), lambda i: (i, 0)),
        ],
        out_shape=[
            jax.ShapeDtypeStruct((N, D), jnp.float32),
            jax.ShapeDtypeStruct((N, D), jnp.float32),
        ],
    )(*args)


def _pool_tc(m, batch2, w1, b1, w2, b2):
    def body(m_ref, batch_ref, w1_ref, b1_ref, w2_ref, b2_ref, emb_ref):
        mm = m_ref[...]
        g1 = jnp.maximum(mm @ w1_ref[...] + b1_ref[...], 0.0)
        gsc = g1 @ w2_ref[...] + b2_ref[...]
        iota = lax.broadcasted_iota(jnp.int32, (N, G), 1)
        oh = (batch_ref[...] == iota).astype(jnp.float32)
        gmax = jnp.max(jnp.where(oh > 0, gsc, -1e30), axis=0, keepdims=True)
        gm_n = lax.dot_general(oh, gmax, (((1,), (1,)), ((), ())))
        exg = jnp.exp(gsc - gm_n)
        deng = lax.dot_general(oh, exg, (((0,), (0,)), ((), ())))
        den_n = lax.dot_general(oh, deng, (((1,), (0,)), ((), ())))
        gate = exg / (den_n + 1e-16)
        emb_ref[...] = lax.dot_general(oh, gate * mm, (((0,), (0,)), ((), ())))

    return pl.pallas_call(
        body,
        out_shape=jax.ShapeDtypeStruct((G, D), jnp.float32),
    )(m, batch2, w1, b1, w2, b2)


def _mlp_tc(emb, y, ws, bs):
    def body(emb_ref, y_ref, w0, w1, w2, w3, b0, b1, b2, b3, preds_ref, tot_ref, los_ref):
        wrefs = [w0, w1, w2, w3]
        brefs = [b0, b1, b2, b3]
        preds = []
        losses = []
        for i in range(6):
            h = emb_ref[...]
            for j in range(4):
                h = h @ wrefs[j][i] + brefs[j][i].reshape(1, -1)
                if j != 3:
                    h = jnp.where(h > 0, h, jnp.exp(h) - 1.0)
            preds.append(h)
            gt = y_ref[:, i:i + 1]
            losses.append(jnp.sqrt(jnp.mean((h - gt) ** 2)))
        preds_ref[...] = jnp.concatenate(preds, axis=1)
        tot_ref[...] = (losses[0] + losses[1] + losses[2]
                        + losses[3] + losses[4] + losses[5]).reshape(1, 1)
        los_ref[...] = jnp.concatenate([l.reshape(1, 1) for l in losses], axis=1)

    return pl.pallas_call(
        body,
        out_shape=[
            jax.ShapeDtypeStruct((G, 6), jnp.float32),
            jax.ShapeDtypeStruct((1, 1), jnp.float32),
            jax.ShapeDtypeStruct((1, 6), jnp.float32),
        ],
    )(emb, y, *ws, *bs)


def kernel(x, edge_index, edge_attr, batch, y, params):
    src = edge_index[0]
    dst = edge_index[1]
    dst3 = dst.reshape(NW, NCH_S, CHS)
    gidx = jnp.concatenate([src, dst + N]).reshape(NW, NCH_G, CH)

    h = x
    m = None
    for li in range(6):
        p = params["convs"][li]
        f = h.shape[1]
        wall = jnp.stack([
            jnp.concatenate([p["k"]["w"], p["v"]["w"]], axis=1),
            jnp.concatenate([p["q"]["w"], jnp.zeros((f, D), jnp.float32)], axis=1),
        ])
        ball = jnp.stack([
            jnp.concatenate([p["k"]["b"], p["v"]["b"]]).reshape(1, 2 * D),
            jnp.concatenate([p["q"]["b"], jnp.zeros((D,), jnp.float32)]).reshape(1, 2 * D),
        ])
        t2 = _tables_tc(h, wall, ball)
        we, be = p["e"]["w"], p["e"]["b"].reshape(1, D)
        gath = _gather_all(t2, gidx)
        msg = _edge_tc(gath, gath, edge_attr, we, be)
        accs = _scatter_add(msg, dst3)
        h, m = _combine_tc(
            accs, h, p["skip"]["w"], p["skip"]["b"].reshape(1, D),
            m, first=(li == 0), apply_elu=(li < 5),
        )

    gp = params["gate"]
    emb = _pool_tc(
        m, batch.reshape(N, 1),
        gp[0]["w"], gp[0]["b"].reshape(1, D), gp[1]["w"], gp[1]["b"].reshape(1, 1),
    )
    ws = [jnp.stack([params["mlps"][i][j]["w"] for i in range(6)]) for j in range(4)]
    bs = [jnp.stack([params["mlps"][i][j]["b"] for i in range(6)]) for j in range(4)]
    preds, tot, losses = _mlp_tc(emb, y, ws, bs)
    return preds, tot[0, 0], losses[0]


# speedup vs baseline: 1.0006x; 1.0006x over previous
import functools

import jax
import jax.numpy as jnp
from jax import lax
from jax.experimental import pallas as pl
from jax.experimental.pallas import tpu as pltpu
from jax.experimental.pallas import tpu_sc as plsc

N = 10000
E = 320000
F_IN = 128
D = 64
EDGE_DIM = 16
G = 64

NC = 2
NS = 16
NW = NC * NS
CH = 80
EPW_G = 2 * E // NW
NCH_G = EPW_G // CH
GBUF = 5
CHS = 80
EPW_S = E // NW
NCH_S = EPW_S // CHS
SBUF = 5
MW = 72
NPAD = 10240
RPS = NPAD // NS
ZR = 8


def _worker_id():
    return lax.axis_index("s") * NC + lax.axis_index("c")


def _make_gather(dt):
    mesh = plsc.VectorSubcoreMesh(core_axis_name="c", subcore_axis_name="s")

    @functools.partial(
        pl.kernel,
        out_type=jax.ShapeDtypeStruct((2 * E, dt), jnp.float32),
        mesh=mesh,
        scratch_types=[
            pltpu.VMEM((NCH_G, CH), jnp.int32),
            pltpu.VMEM((GBUF, CH, dt), jnp.float32),
            pltpu.SemaphoreType.DMA((GBUF,)),
        ],
    )
    def gather_k(table_hbm, idx_hbm, out_hbm, idx_v, rows_v, sems):
        w = _worker_id()
        pltpu.sync_copy(idx_hbm.at[w], idx_v)
        ebase = w * EPW_G
        for b in range(GBUF):
            pltpu.make_async_copy(
                table_hbm.at[idx_v.at[b]], rows_v.at[b], sems.at[b]
            ).start()

        def body(t, _):
            for b in range(GBUF):
                i = t * GBUF + b
                pltpu.make_async_copy(
                    table_hbm.at[idx_v.at[i]], rows_v.at[b], sems.at[b]
                ).wait()
                pltpu.sync_copy(rows_v.at[b], out_hbm.at[pl.ds(ebase + i * CH, CH)])
                nxt = i + GBUF

                @pl.when(nxt < NCH_G)
                def _start():
                    pltpu.make_async_copy(
                        table_hbm.at[idx_v.at[nxt]], rows_v.at[b], sems.at[b]
                    ).start()

            return 0

        lax.fori_loop(0, NCH_G // GBUF, body, 0)

    return gather_k


def _make_scatter_add():
    mesh = plsc.VectorSubcoreMesh(core_axis_name="c", subcore_axis_name="s")

    @functools.partial(
        pl.kernel,
        out_type=jax.ShapeDtypeStruct((NC, NPAD, MW), jnp.float32),
        mesh=mesh,
        scratch_types=[
            pltpu.VMEM_SHARED((NPAD, MW), jnp.float32),
            pltpu.VMEM((NCH_S, CHS), jnp.int32),
            pltpu.VMEM((SBUF, CHS, MW), jnp.float32),
            pltpu.VMEM((ZR, MW), jnp.float32),
            pltpu.SemaphoreType.DMA((SBUF,)),
        ],
    )
    def scatter_k(msg_hbm, dst_hbm, out_hbm, acc_sh, dst_v, msg_v, zbuf, sems):
        c = lax.axis_index("c")
        s = lax.axis_index("s")
        w = s * NC + c

        def zrow(r, _):
            for cc in range(MW // 16):
                zbuf[r, pl.ds(cc * 16, 16)] = jnp.zeros((16,), jnp.float32)
            return 0

        lax.fori_loop(0, ZR, zrow, 0)
        for j in range(RPS // ZR):
            pltpu.sync_copy(zbuf, acc_sh.at[pl.ds(s * RPS + j * ZR, ZR)])
        plsc.subcore_barrier()

        pltpu.sync_copy(dst_hbm.at[w], dst_v)
        for b in range(SBUF):
            pltpu.make_async_copy(
                msg_hbm.at[pl.ds(w * EPW_S + b * CHS, CHS)], msg_v.at[b], sems.at[b]
            ).start()

        def body(t, _):
            for b in range(SBUF):
                i = t * SBUF + b
                pltpu.make_async_copy(
                    msg_hbm.at[pl.ds(w * EPW_S + i * CHS, CHS)], msg_v.at[b], sems.at[b]
                ).wait()
                pltpu.sync_copy(msg_v.at[b], acc_sh.at[dst_v.at[i]], add=True)
                nxt = i + SBUF

                @pl.when(nxt < NCH_S)
                def _start():
                    pltpu.make_async_copy(
                        msg_hbm.at[pl.ds(w * EPW_S + nxt * CHS, CHS)],
                        msg_v.at[b],
                        sems.at[b],
                    ).start()

            return 0

        lax.fori_loop(0, NCH_S // SBUF, body, 0)
        plsc.subcore_barrier()
        pltpu.sync_copy(acc_sh.at[pl.ds(s * RPS, RPS)], out_hbm.at[c, pl.ds(s * RPS, RPS)])

    return scatter_k


_gather_all = _make_gather(2 * D)
_scatter_add = _make_scatter_add()


_BN = 1000
_BEDGE = 4000


def _tables_tc(h, wall, ball):
    f = h.shape[1]
    nb = N // _BN

    def body(h_ref, w_ref, b_ref, out_ref):
        out_ref[...] = h_ref[...] @ w_ref[0] + b_ref[0]

    return pl.pallas_call(
        body,
        grid=(2 * nb,),
        in_specs=[
            pl.BlockSpec((_BN, f), lambda i: (i % nb, 0)),
            pl.BlockSpec((1, f, 2 * D), lambda i: (i // nb, 0, 0)),
            pl.BlockSpec((1, 1, 2 * D), lambda i: (i // nb, 0, 0)),
        ],
        out_specs=pl.BlockSpec((_BN, 2 * D), lambda i: (i, 0)),
        out_shape=jax.ShapeDtypeStruct((2 * N, 2 * D), jnp.float32),
    )(h, wall, ball)


def _edge_tc(kvj, qd, ea, we, be):
    def body(kvj_ref, qd_ref, ea_ref, we_ref, be_ref, out_ref):
        e = ea_ref[...] @ we_ref[...] + be_ref[...]
        kj = kvj_ref[:, :D] + e
        alpha = jnp.sum(qd_ref[:, :D] * kj, axis=1, keepdims=True) * 0.125
        ex = jnp.exp(alpha)
        out_ref[:, :D] = (kvj_ref[:, D:] + e) * ex
        out_ref[:, D:D + 1] = ex
        out_ref[:, D + 1:] = jnp.zeros((out_ref.shape[0], MW - D - 1), jnp.float32)

    nbe = E // _BEDGE
    return pl.pallas_call(
        body,
        grid=(nbe,),
        in_specs=[
            pl.BlockSpec((_BEDGE, 2 * D), lambda i: (i, 0)),
            pl.BlockSpec((_BEDGE, 2 * D), lambda i: (i + nbe, 0)),
            pl.BlockSpec((_BEDGE, EDGE_DIM), lambda i: (i, 0)),
            pl.BlockSpec((EDGE_DIM, D), lambda i: (0, 0)),
            pl.BlockSpec((1, D), lambda i: (0, 0)),
        ],
        out_specs=pl.BlockSpec((_BEDGE, MW), lambda i: (i, 0)),
        out_shape=jax.ShapeDtypeStruct((E, MW), jnp.float32),
    )(kvj, qd, ea, we, be)


def _combine_tc(accs, h, ws, bs, m_prev, first, apply_elu):
    f = h.shape[1]

    def body(*refs):
        if first:
            acc_ref, h_ref, ws_ref, bs_ref, h_out, m_out = refs
        else:
            acc_ref, h_ref, ws_ref, bs_ref, m_ref, h_out, m_out = refs
        acc = acc_ref[0] + acc_ref[1]
        den = acc[:, D:D + 1]
        conv = acc[:, :D] / (den + 1e-16) + h_ref[...] @ ws_ref[...] + bs_ref[...]
        if apply_elu:
            conv = jnp.where(conv > 0, conv, jnp.exp(conv) - 1.0)
        h_out[...] = conv
        if first:
            m_out[...] = conv
        else:
            m_out[...] = jnp.maximum(m_ref[...], conv)

    in_specs = [
        pl.BlockSpec((NC, _BN, MW), lambda i: (0, i, 0)),
        pl.BlockSpec((_BN, f), lambda i: (i, 0)),
        pl.BlockSpec((f, D), lambda i: (0, 0)),
        pl.BlockSpec((1, D), lambda i: (0, 0)),
    ]
    args = [accs, h, ws, bs]
    if not first:
        in_specs.append(pl.BlockSpec((_BN, D), lambda i: (i, 0)))
        args.append(m_prev)

    return pl.pallas_call(
        body,
        grid=(N // _BN,),
        in_specs=in_specs,
        out_specs=[
            pl.BlockSpec((_BN, D), lambda i: (i, 0)),
            pl.BlockSpec((_BN, D), lambda i: (i, 0)),
        ],
        out_shape=[
            jax.ShapeDtypeStruct((N, D), jnp.float32),
            jax.ShapeDtypeStruct((N, D), jnp.float32),
        ],
    )(*args)


def _pool_tc(m, batch2, w1, b1, w2, b2):
    def body(m_ref, batch_ref, w1_ref, b1_ref, w2_ref, b2_ref, emb_ref):
        mm = m_ref[...]
        g1 = jnp.maximum(mm @ w1_ref[...] + b1_ref[...], 0.0)
        gsc = g1 @ w2_ref[...] + b2_ref[...]
        iota = lax.broadcasted_iota(jnp.int32, (N, G), 1)
        oh = (batch_ref[...] == iota).astype(jnp.float32)
        gmax = jnp.max(jnp.where(oh > 0, gsc, -1e30), axis=0, keepdims=True)
        gm_n = lax.dot_general(oh, gmax, (((1,), (1,)), ((), ())))
        exg = jnp.exp(gsc - gm_n)
        deng = lax.dot_general(oh, exg, (((0,), (0,)), ((), ())))
        den_n = lax.dot_general(oh, deng, (((1,), (0,)), ((), ())))
        gate = exg / (den_n + 1e-16)
        emb_ref[...] = lax.dot_general(oh, gate * mm, (((0,), (0,)), ((), ())))

    return pl.pallas_call(
        body,
        out_shape=jax.ShapeDtypeStruct((G, D), jnp.float32),
    )(m, batch2, w1, b1, w2, b2)


def _mlp_tc(emb, y, ws, bs):
    def body(emb_ref, y_ref, w0, w1, w2, w3, b0, b1, b2, b3, preds_ref, tot_ref, los_ref):
        wrefs = [w0, w1, w2, w3]
        brefs = [b0, b1, b2, b3]
        preds = []
        losses = []
        for i in range(6):
            h = emb_ref[...]
            for j in range(4):
                h = h @ wrefs[j][i] + brefs[j][i].reshape(1, -1)
                if j != 3:
                    h = jnp.where(h > 0, h, jnp.exp(h) - 1.0)
            preds.append(h)
            gt = y_ref[:, i:i + 1]
            losses.append(jnp.sqrt(jnp.mean((h - gt) ** 2)))
        preds_ref[...] = jnp.concatenate(preds, axis=1)
        tot_ref[...] = (losses[0] + losses[1] + losses[2]
                        + losses[3] + losses[4] + losses[5]).reshape(1, 1)
        los_ref[...] = jnp.concatenate([l.reshape(1, 1) for l in losses], axis=1)

    return pl.pallas_call(
        body,
        out_shape=[
            jax.ShapeDtypeStruct((G, 6), jnp.float32),
            jax.ShapeDtypeStruct((1, 1), jnp.float32),
            jax.ShapeDtypeStruct((1, 6), jnp.float32),
        ],
    )(emb, y, *ws, *bs)


def kernel(x, edge_index, edge_attr, batch, y, params):
    src = edge_index[0]
    dst = edge_index[1]
    dst3 = dst.reshape(NW, NCH_S, CHS)
    gidx = jnp.concatenate([src, dst + N]).reshape(NW, NCH_G, CH)

    h = x
    m = None
    for li in range(6):
        p = params["convs"][li]
        f = h.shape[1]
        wall = jnp.stack([
            jnp.concatenate([p["k"]["w"], p["v"]["w"]], axis=1),
            jnp.concatenate([p["q"]["w"], jnp.zeros((f, D), jnp.float32)], axis=1),
        ])
        ball = jnp.stack([
            jnp.concatenate([p["k"]["b"], p["v"]["b"]]).reshape(1, 2 * D),
            jnp.concatenate([p["q"]["b"], jnp.zeros((D,), jnp.float32)]).reshape(1, 2 * D),
        ])
        t2 = _tables_tc(h, wall, ball)
        we, be = p["e"]["w"], p["e"]["b"].reshape(1, D)
        gath = _gather_all(t2, gidx)
        msg = _edge_tc(gath, gath, edge_attr, we, be)
        accs = _scatter_add(msg, dst3)
        h, m = _combine_tc(
            accs, h, p["skip"]["w"], p["skip"]["b"].reshape(1, D),
            m, first=(li == 0), apply_elu=(li < 5),
        )

    gp = params["gate"]
    emb = _pool_tc(
        m, batch.reshape(N, 1),
        gp[0]["w"], gp[0]["b"].reshape(1, D), gp[1]["w"], gp[1]["b"].reshape(1, 1),
    )
    ws = [jnp.stack([params["mlps"][i][j]["w"] for i in range(6)]) for j in range(4)]
    bs = [jnp.stack([params["mlps"][i][j]["b"] for i in range(6)]) for j in range(4)]
    preds, tot, losses = _mlp_tc(emb, y, ws, bs)
    return preds, tot[0, 0], losses[0]
